# baseline (speedup 1.0000x reference)
import jax
import jax.numpy as jnp
from jax.experimental import pallas as pl

_N_ETYPES = 2
_N_LAYERS = 2
_N_FIELDS = 4
_NUM_DST = 10000


def _copy_body(x_ref, o_ref):
    o_ref[...] = x_ref[...]


def _pallas_touch(x):
    return pl.pallas_call(
        _copy_body,
        out_shape=jax.ShapeDtypeStruct(x.shape, x.dtype),
    )(x)


def _gat_conv(h, src, dst, W, al, ar):
    n = h.shape[0]
    sl = jnp.arange(n, dtype=src.dtype)
    s = jnp.concatenate([src, sl])
    d = jnp.concatenate([dst, sl])
    z = h @ W
    el = z @ al
    er = z @ ar
    e = jax.nn.leaky_relu(el[s] + er[d], negative_slope=0.2)
    emax = jax.ops.segment_max(e, d, num_segments=n)
    ee = jnp.exp(e - emax[d])
    denom = jax.ops.segment_sum(ee, d, num_segments=n)
    alpha = ee / (denom[d] + 1e-9)
    out = jax.ops.segment_sum(alpha[:, None] * z[s], d, num_segments=n)
    return jax.nn.elu(out)


def _semantic_attention(z3, W1, b1, W2):
    w = jnp.tanh(z3 @ W1 + b1) @ W2
    w = w.mean(axis=0)
    beta = jax.nn.softmax(w, axis=0)
    return (z3 * beta[None, :, None]).sum(axis=1)


def kernel(x, edge_index_b0_e0, edge_index_b0_e1, edge_index_b1_e0, edge_index_b1_e1, num_dst, params):
    blocks = [
        [(edge_index_b0_e0[0], edge_index_b0_e0[1]), (edge_index_b0_e1[0], edge_index_b0_e1[1])],
        [(edge_index_b1_e0[0], edge_index_b1_e0[1]), (edge_index_b1_e1[0], edge_index_b1_e1[1])],
    ]
    feats = jnp.concatenate([params['emb_%d' % f][x[:, f]] for f in range(_N_FIELDS)], axis=1)
    outs = []
    for e in range(_N_ETYPES):
        h = feats @ params['fc_in_W_%d' % e] + params['fc_in_b_%d' % e]
        for l in range(_N_LAYERS):
            sems = []
            for m in range(_N_ETYPES):
                src, dst = blocks[l][m]
                sems.append(_gat_conv(h, src, dst,
                                      params['gat_W_%d_%d_%d' % (e, l, m)],
                                      params['gat_al_%d_%d_%d' % (e, l, m)],
                                      params['gat_ar_%d_%d_%d' % (e, l, m)]))
            z3 = jnp.stack(sems, axis=1)
            h = _semantic_attention(z3, params['sem_W1_%d_%d' % (e, l)], params['sem_b1_%d_%d' % (e, l)], params['sem_W2_%d_%d' % (e, l)])
            if l != _N_LAYERS - 1:
                h = jax.nn.relu(h)
            else:
                keep = jnp.arange(_NUM_DST) < num_dst
                h = jnp.where(keep[:, None], h[:_NUM_DST], jnp.zeros((), dtype=h.dtype))
                h = h @ params['fc_out_W_%d' % e] + params['fc_out_b_%d' % e]
        outs.append(h)
    out = jnp.stack(outs, axis=0)
    return _pallas_touch(out)


# speedup vs baseline: 27.8448x; 27.8448x over previous
import jax
import jax.numpy as jnp
from jax import lax
from jax.experimental import pallas as pl
from jax.experimental.pallas import tpu as pltpu
from jax.experimental.pallas import tpu_sc as plsc

F32 = jnp.float32
I32 = jnp.int32
N = 50000
E = 800000
HID = 64
BN = 1000
NB = N // BN
NEG = 0.2


def _mm_bias_body(xr, wr, br, orf):
    orf[...] = jnp.dot(xr[...], wr[...], preferred_element_type=F32) + br[...]


def _tc_mm_bias(xx, w, b):
    n, k = xx.shape
    m = w.shape[1]
    return pl.pallas_call(
        _mm_bias_body,
        grid=(n // BN,),
        in_specs=[pl.BlockSpec((BN, k), lambda i: (i, 0)),
                  pl.BlockSpec((k, m), lambda i: (0, 0)),
                  pl.BlockSpec((1, m), lambda i: (0, 0))],
        out_specs=pl.BlockSpec((BN, m), lambda i: (i, 0)),
        out_shape=jax.ShapeDtypeStruct((n, m), F32),
    )(xx, w, b.reshape(1, m))


def _pre_body(hr, wr, alr, arr, zxr, melr):
    i = pl.program_id(0)
    z = jnp.dot(hr[...], wr[...], preferred_element_type=F32)
    el = jnp.dot(z, alr[...], preferred_element_type=F32)
    er = jnp.dot(z, arr[...], preferred_element_type=F32)
    zxr[...] = jnp.concatenate([z, el, er, jnp.zeros((BN, 62), F32)], axis=1)
    bm = jnp.max(el)

    @pl.when(i == 0)
    def _():
        melr[...] = jnp.full((1, 1), bm, F32)

    @pl.when(i > 0)
    def _():
        melr[...] = jnp.maximum(melr[...], bm)


def _tc_pre(h, w, al, ar):
    return pl.pallas_call(
        _pre_body,
        grid=(NB,),
        in_specs=[pl.BlockSpec((BN, HID), lambda i: (i, 0)),
                  pl.BlockSpec((HID, HID), lambda i: (0, 0)),
                  pl.BlockSpec((HID, 1), lambda i: (0, 0)),
                  pl.BlockSpec((HID, 1), lambda i: (0, 0))],
        out_specs=[pl.BlockSpec((BN, 128), lambda i: (i, 0)),
                   pl.BlockSpec((1, 1), lambda i: (0, 0))],
        out_shape=[jax.ShapeDtypeStruct((N, 128), F32),
                   jax.ShapeDtypeStruct((1, 1), F32)],
    )(h, w, al, ar)


def _make_fin_body(e):
    def _fin_body(ar1, str_, zxr, melr, orf):
        mel = melr[0, 0]
        zx = zxr[...]
        el = zx[:, 64:65]
        er = zx[:, 65:66]
        u = el - mel
        t = el + er
        p = t > 0.0
        ws = jnp.exp(jnp.where(p, u, NEG * u))
        cm = mel + er
        cc = jnp.where(cm > 0.0, cm, NEG * cm)
        f1 = jnp.exp(er + mel - cc)
        f2 = jnp.exp(NEG * (er + mel) - cc)
        w1 = jnp.where(p, ws, 0.0)
        w2 = jnp.where(p, 0.0, ws)
        st = str_[...]
        s1 = st[:, 2 * e:2 * e + 1] + w1
        s2 = st[:, 2 * e + 1:2 * e + 2] + w2
        den = f1 * s1 + f2 * s2 + 1e-9
        a = ar1[0]
        cols = []
        for q in range(4):
            zq = zx[:, 16 * q:16 * q + 16]
            a1 = a[:, 16 * q:16 * q + 16] + w1 * zq
            a2 = a[:, 64 + 16 * q:64 + 16 * q + 16] + w2 * zq
            cols.append((f1 * a1 + f2 * a2) / den)
        o = jnp.concatenate(cols, axis=1)
        orf[...] = jnp.where(o > 0.0, o, jnp.exp(jnp.minimum(o, 0.0)) - 1.0)
    return _fin_body


def _tc_fin(e, a, st, zx, mel):
    return pl.pallas_call(
        _make_fin_body(e),
        grid=(NB,),
        in_specs=[pl.BlockSpec((1, BN, 128), lambda i, e=e: (e, i, 0)),
                  pl.BlockSpec((BN, 4), lambda i: (i, 0)),
                  pl.BlockSpec((BN, 128), lambda i: (i, 0)),
                  pl.BlockSpec((1, 1), lambda i: (0, 0))],
        out_specs=pl.BlockSpec((BN, HID), lambda i: (i, 0)),
        out_shape=jax.ShapeDtypeStruct((N, HID), F32),
    )(a, st, zx, mel)


def _sema_body(h0r, h1r, w1r, b1r, w2r, orf):
    i = pl.program_id(0)
    lane = lax.broadcasted_iota(I32, (1, 128), 1)
    acc = jnp.zeros((1, 128), F32)
    for m, hr in ((0, h0r), (1, h1r)):
        tt = jnp.tanh(jnp.dot(hr[...], w1r[...], preferred_element_type=F32) + b1r[...])
        wv = jnp.dot(tt, w2r[...], preferred_element_type=F32)
        acc = acc + jnp.where(lane == m, jnp.sum(wv), 0.0)

    @pl.when(i == 0)
    def _():
        orf[...] = jnp.zeros_like(orf)

    orf[...] += acc


def _tc_sem_a(h0, h1, w1, b1, w2):
    return pl.pallas_call(
        _sema_body,
        grid=(NB,),
        in_specs=[pl.BlockSpec((BN, HID), lambda i: (i, 0)),
                  pl.BlockSpec((BN, HID), lambda i: (i, 0)),
                  pl.BlockSpec((HID, 128), lambda i: (0, 0)),
                  pl.BlockSpec((1, 128), lambda i: (0, 0)),
                  pl.BlockSpec((128, 1), lambda i: (0, 0))],
        out_specs=pl.BlockSpec((1, 128), lambda i: (0, 0)),
        out_shape=jax.ShapeDtypeStruct((1, 128), F32),
    )(h0, h1, w1, b1, w2)


def _semb_body_relu(sr, h0r, h1r, orf):
    _semb_common(sr, h0r, h1r, orf, True)


def _semb_body_plain(sr, h0r, h1r, orf):
    _semb_common(sr, h0r, h1r, orf, False)


def _semb_common(sr, h0r, h1r, orf, relu):
    s0 = sr[0, 0] / N
    s1 = sr[0, 1] / N
    mx = jnp.maximum(s0, s1)
    b0 = jnp.exp(s0 - mx)
    b1 = jnp.exp(s1 - mx)
    beta0 = b0 / (b0 + b1)
    o = beta0 * h0r[...] + (1.0 - beta0) * h1r[...]
    if relu:
        o = jnp.maximum(o, 0.0)
    orf[...] = o


def _tc_sem_b(sums, h0, h1, relu):
    return pl.pallas_call(
        _semb_body_relu if relu else _semb_body_plain,
        grid=(NB,),
        in_specs=[pl.BlockSpec((1, 128), lambda i: (0, 0)),
                  pl.BlockSpec((BN, HID), lambda i: (i, 0)),
                  pl.BlockSpec((BN, HID), lambda i: (i, 0))],
        out_specs=pl.BlockSpec((BN, HID), lambda i: (i, 0)),
        out_shape=jax.ShapeDtypeStruct((N, HID), F32),
    )(sums, h0, h1)


def _out_body(hr, wr, br, ndr, orf):
    i = pl.program_id(0)
    rows = i * BN + lax.broadcasted_iota(I32, (BN, 1), 0)
    keep = rows < ndr[0, 0]
    hm = jnp.where(keep, hr[...], 0.0)
    orf[...] = jnp.dot(hm, wr[...], preferred_element_type=F32) + br[...]


def _tc_out(h, w, b, nd, num_out):
    return pl.pallas_call(
        _out_body,
        grid=(num_out // BN,),
        in_specs=[pl.BlockSpec((BN, HID), lambda i: (i, 0)),
                  pl.BlockSpec((HID, HID), lambda i: (0, 0)),
                  pl.BlockSpec((1, HID), lambda i: (0, 0)),
                  pl.BlockSpec((1, 1), lambda i: (0, 0))],
        out_specs=pl.BlockSpec((BN, HID), lambda i: (i, 0)),
        out_shape=jax.ShapeDtypeStruct((num_out, HID), F32),
    )(h, w, b, nd)


def _mesh():
    return plsc.VectorSubcoreMesh(core_axis_name="c", subcore_axis_name="s",
                                  num_cores=2, num_subcores=16)

_ECH = 640
_NECH = E // _ECH
_EK = -(-_NECH // 16)
_ZCH = 32
_NZCH = -(-N // _ZCH)
_RZ = 256
_NRZ = -(-(2 * N) // _RZ)
_ACH = 500
_SCH = 400


def _conv_body(ei_h, el2_h, er2_h, mel2_h, zx_h, zz_h, a_h, s_h, si_h, di_h,
               mel_t, sb, db, elsb, erdb, sib, dib, wb, rows, zb,
               zxb, zzb1, zzb2,
               el_sh, er_sh, a_sh, s_sh, semA, semS, semZ):
    c = lax.axis_index("c")
    sid = lax.axis_index("s")
    pltpu.sync_copy(mel2_h.at[c], mel_t)

    @pl.when(sid == 0)
    def _():
        pltpu.sync_copy(el2_h.at[c], el_sh)
        pltpu.sync_copy(er2_h.at[c], er_sh)

    melv = mel_t[...]
    z16 = jnp.zeros((16,), F32)
    lanes = lax.broadcasted_iota(I32, (16,), 0)
    for j in range(_SCH // 16):
        zb[pl.ds(j * 16, 16)] = z16

    def zs(k, _):
        cid = sid + 16 * k

        @pl.when(cid < (2 * N) // _SCH)
        def _():
            pltpu.sync_copy(zb, s_sh.at[pl.ds(pl.multiple_of(cid * _SCH, 8), _SCH)])
        return 0

    lax.fori_loop(0, -(-((2 * N) // _SCH) // 16), zs, 0)

    def zchunk(k, _):
        cid = sid + 16 * k

        @pl.when(cid < _NZCH)
        def _():
            base = pl.multiple_of(jnp.minimum(cid * _ZCH, N - _ZCH), 8)
            pltpu.sync_copy(zx_h.at[c, pl.ds(base, _ZCH)], zxb)
            w1v, w2v = [], []
            for g in range(_ZCH // 16):
                el16 = plsc.load_gather(zxb, [g * 16 + lanes, jnp.full((16,), 64, I32)])
                u = el16 - melv
                w1v.append(jnp.exp(u))
                w2v.append(jnp.exp(NEG * u))

            @pl.when(k > 0)
            def _():
                for _i in range(8):
                    pltpu.make_async_copy(zz_h.at[pl.ds(0, _ZCH)],
                                          zzb1.at[pl.ds(0, _ZCH), pl.ds(0, 16)],
                                          semZ).wait()

            for j in range(_ZCH):
                g, j0 = j // 16, j % 16
                wv1 = w1v[g][j0]
                wv2 = w2v[g][j0]
                for q in range(4):
                    v = zxb[j, pl.ds(16 * q, 16)]
                    zzb1[j, pl.ds(16 * q, 16)] = v * wv1
                    zzb2[j, pl.ds(16 * q, 16)] = v * wv2
            for b, zzb in ((0, zzb1), (1, zzb2)):
                for q in range(4):
                    pltpu.async_copy(
                        zzb.at[pl.ds(0, _ZCH), pl.ds(16 * q, 16)],
                        zz_h.at[pl.ds((c * 8 + b * 4 + q) * N + base, _ZCH)],
                        semZ)
        return 0

    lax.fori_loop(0, -(-_NZCH // 16), zchunk, 0)
    for _i in range(8):
        pltpu.make_async_copy(zz_h.at[pl.ds(0, _ZCH)],
                              zzb1.at[pl.ds(0, _ZCH), pl.ds(0, 16)], semZ).wait()
    plsc.subcore_barrier()

    for q in range(4):
        def zrow(i, _):
            rows[i, :] = z16
            return 0

        lax.fori_loop(0, _RZ, zrow, 0)

        def za(k, _):
            cid = sid + 16 * k

            @pl.when(cid < _NRZ)
            def _():
                base = jnp.minimum(cid * _RZ, 2 * N - _RZ)
                pltpu.sync_copy(rows.at[pl.ds(0, _RZ)], a_sh.at[pl.ds(base, _RZ)])
            return 0

        lax.fori_loop(0, -(-_NRZ // 16), za, 0)
        plsc.subcore_barrier()

        def ech(k, _):
            cid = sid + 16 * k

            @pl.when(cid < _NECH)
            def _():
                pend = []
                if q == 0:
                    h1 = pltpu.async_copy(ei_h.at[0, cid], sb, semA)
                    h2 = pltpu.async_copy(ei_h.at[1, cid], db, semA)
                    h1.wait()
                    h2.wait()
                    ghs = []
                    for i in range(5):
                        ghs.append(pltpu.async_copy(el_sh.at[sb.at[i]], elsb.at[i], semA))
                        ghs.append(pltpu.async_copy(er_sh.at[db.at[i]], erdb.at[i], semA))
                    for h in ghs:
                        h.wait()
                    for j in range(_ECH // 16):
                        r, cs = j // 8, (j % 8) * 16
                        s16 = sb[r, pl.ds(cs, 16)]
                        d16 = db[r, pl.ds(cs, 16)]
                        els = elsb[r, pl.ds(cs, 16)]
                        erd = erdb[r, pl.ds(cs, 16)]
                        tv = els + erd
                        p = tv > 0.0
                        si = s16 + c * (8 * N) + jnp.where(p, 0, 4 * N).astype(I32)
                        di = d16 + jnp.where(p, 0, N).astype(I32)
                        sib[r, pl.ds(cs, 16)] = si
                        dib[r, pl.ds(cs, 16)] = di
                        u = els - melv
                        wb[pl.ds(j * 16, 16)] = jnp.exp(jnp.where(p, u, NEG * u))
                    pend.append(pltpu.async_copy(sib, si_h.at[c, cid], semS))
                    pend.append(pltpu.async_copy(dib, di_h.at[c, cid], semS))
                else:
                    h1 = pltpu.async_copy(si_h.at[c, cid], sb, semA)
                    h2 = pltpu.async_copy(di_h.at[c, cid], dib, semA)
                    h1.wait()
                    h2.wait()
                    for j in range(_ECH // 16):
                        r, cs = j // 8, (j % 8) * 16
                        sib[r, pl.ds(cs, 16)] = sb[r, pl.ds(cs, 16)] + q * N

                gh = [None] * 5
                sh = [None] * 5
                gh[0] = pltpu.async_copy(zz_h.at[sib.at[0]],
                                         rows.at[pl.ds(0, 128)], semA)
                for i in range(5):
                    if i + 1 < 5:
                        if i - 1 >= 0:
                            sh[i - 1].wait()
                        gh[i + 1] = pltpu.async_copy(
                            zz_h.at[sib.at[i + 1]],
                            rows.at[pl.ds(128 * ((i + 1) % 2), 128)], semA)
                    gh[i].wait()
                    sh[i] = pltpu.async_copy(rows.at[pl.ds(128 * (i % 2), 128)],
                                             a_sh.at[dib.at[i]], semS, add=True)
                    if q == 0:
                        pend.append(pltpu.async_copy(wb.at[pl.ds(128 * i, 128)],
                                                     s_sh.at[dib.at[i]], semS, add=True))
                for i in (3, 4):
                    sh[i].wait()
                for h in pend:
                    h.wait()
            return 0

        lax.fori_loop(0, _EK, ech, 0)
        plsc.subcore_barrier()

        def da(k, _):
            cid = sid + 16 * k

            @pl.when(cid < 2 * (N // _ACH))
            def _():
                br = cid // (N // _ACH)
                nb = cid - br * (N // _ACH)
                pltpu.sync_copy(
                    a_sh.at[pl.ds(br * N + nb * _ACH, _ACH)],
                    a_h.at[c, pl.ds(nb * _ACH, _ACH), pl.ds(64 * br + 16 * q, 16)])
            return 0

        lax.fori_loop(0, -(-(2 * (N // _ACH)) // 16), da, 0)
        if q == 0:
            def dsm(k, _):
                cid = sid + 16 * k

                @pl.when(cid < (2 * N) // _SCH)
                def _():
                    b2 = pl.multiple_of(cid * _SCH, 8)
                    pltpu.sync_copy(s_sh.at[pl.ds(b2, _SCH)], s_h.at[c, pl.ds(b2, _SCH)])
                return 0

            lax.fori_loop(0, -(-((2 * N) // _SCH) // 16), dsm, 0)
        plsc.subcore_barrier()


def _sc_conv(ei, el2, er2, mel2, zx2):
    ei = ei.reshape(2, _NECH, 5, 128)
    return pl.kernel(
        _conv_body,
        out_type=(jax.ShapeDtypeStruct((16 * N, 16), F32),
                  jax.ShapeDtypeStruct((2, N, 128), F32),
                  jax.ShapeDtypeStruct((2, 2 * N), F32),
                  jax.ShapeDtypeStruct((2, _NECH, 5, 128), I32),
                  jax.ShapeDtypeStruct((2, _NECH, 5, 128), I32)),
        mesh=_mesh(),
        scratch_types=[
            pltpu.VMEM((16,), F32),
            pltpu.VMEM((5, 128), I32),
            pltpu.VMEM((5, 128), I32),
            pltpu.VMEM((5, 128), F32),
            pltpu.VMEM((5, 128), F32),
            pltpu.VMEM((5, 128), I32),
            pltpu.VMEM((5, 128), I32),
            pltpu.VMEM((_ECH,), F32),
            pltpu.VMEM((_RZ, 16), F32),
            pltpu.VMEM((_SCH,), F32),
            pltpu.VMEM((_ZCH, 128), F32),
            pltpu.VMEM((_ZCH, 64), F32),
            pltpu.VMEM((_ZCH, 64), F32),
            pltpu.VMEM_SHARED((N,), F32),
            pltpu.VMEM_SHARED((N,), F32),
            pltpu.VMEM_SHARED((2 * N, 16), F32),
            pltpu.VMEM_SHARED((2 * N,), F32),
            pltpu.SemaphoreType.DMA,
            pltpu.SemaphoreType.DMA,
            pltpu.SemaphoreType.DMA,
        ],
        compiler_params=pltpu.CompilerParams(use_tc_tiling_on_sc=False, needs_layout_passes=False),
    )(ei, el2, er2, mel2, zx2)


_XCH = 128
_NXCH = -(-N // _XCH)
_EMB_DIMS = (16, 16, 16, 32)
_EMB_COLS = (0, 16, 32, 48)


def _embed_body(xt_h, e0_h, e1_h, e2_h, e3_h, feats_h, t0, t1, t2, t3, xb, ft):
    c = lax.axis_index("c")
    sid = lax.axis_index("s")
    wid = c * 16 + sid
    pltpu.sync_copy(e0_h, t0)
    pltpu.sync_copy(e1_h, t1)
    pltpu.sync_copy(e2_h, t2)
    pltpu.sync_copy(e3_h.at[pl.ds(0, 1000)], t3)
    lanes = lax.broadcasted_iota(I32, (16,), 0)
    tabs = (t0, t1, t2, t3)

    def chunk(k, _):
        cid = wid + 32 * k

        @pl.when(cid < _NXCH)
        def _():
            base = pl.multiple_of(jnp.minimum(cid * _XCH, N - _XCH), 8)
            for f in range(4):
                pltpu.sync_copy(xt_h.at[f, pl.ds(base, _XCH)], xb.at[f])
            for j in range(_XCH // 16):
                rowv = j * 16 + lanes
                for f in range(4):
                    idx = xb[f, pl.ds(j * 16, 16)]
                    for kk in range(_EMB_DIMS[f]):
                        v = plsc.load_gather(tabs[f], [idx, jnp.full((16,), kk, I32)])
                        plsc.store_scatter(ft, [rowv, jnp.full((16,), _EMB_COLS[f] + kk, I32)], v)
            pltpu.sync_copy(ft, feats_h.at[pl.ds(base, _XCH)])
        return 0

    lax.fori_loop(0, -(-_NXCH // 32), chunk, 0)


def _sc_embed(xt, e0, e1, e2, e3):
    return pl.kernel(
        _embed_body,
        out_type=jax.ShapeDtypeStruct((N, 80), F32),
        mesh=_mesh(),
        scratch_types=[
            pltpu.VMEM((1000, 16), F32),
            pltpu.VMEM((1000, 16), F32),
            pltpu.VMEM((1000, 16), F32),
            pltpu.VMEM((1000, 32), F32),
            pltpu.VMEM((4, _XCH), I32),
            pltpu.VMEM((_XCH, 80), F32),
        ],
        compiler_params=pltpu.CompilerParams(use_tc_tiling_on_sc=False, needs_layout_passes=False),
    )(xt, e0, e1, e2, e3)


def kernel(x, edge_index_b0_e0, edge_index_b0_e1, edge_index_b1_e0, edge_index_b1_e1, num_dst, params):
    graphs = [[edge_index_b0_e0, edge_index_b0_e1], [edge_index_b1_e0, edge_index_b1_e1]]
    feats = _sc_embed(x.T, params['emb_0'], params['emb_1'], params['emb_2'], params['emb_3'])
    nd = jnp.asarray(num_dst, I32).reshape(1, 1)
    hs = [_tc_mm_bias(feats, params['fc_in_W_%d' % e], params['fc_in_b_%d' % e])
          for e in range(2)]
    for l in range(2):
        convs = [[None, None], [None, None]]
        for m in range(2):
            zxs, mels = [], []
            for e in range(2):
                zx, mel = _tc_pre(
                    hs[e],
                    params['gat_W_%d_%d_%d' % (e, l, m)],
                    params['gat_al_%d_%d_%d' % (e, l, m)].reshape(HID, 1),
                    params['gat_ar_%d_%d_%d' % (e, l, m)].reshape(HID, 1))
                zxs.append(zx)
                mels.append(mel)
            zx2 = jnp.stack(zxs)
            el2 = jnp.stack([zxs[0][:, 64], zxs[1][:, 64]])
            er2 = jnp.stack([zxs[0][:, 65], zxs[1][:, 65]])
            mel2 = jnp.concatenate([jnp.broadcast_to(mels[0], (1, 16)),
                                    jnp.broadcast_to(mels[1], (1, 16))], axis=0)
            _zz, aa, ss, _si, _di = _sc_conv(graphs[l][m], el2, er2, mel2, zx2)
            st = ss.reshape(4, N).T
            for e in range(2):
                convs[e][m] = _tc_fin(e, aa, st, zxs[e], mels[e])
        newhs = []
        for e in range(2):
            sums = _tc_sem_a(convs[e][0], convs[e][1],
                             params['sem_W1_%d_%d' % (e, l)],
                             params['sem_b1_%d_%d' % (e, l)].reshape(1, 128),
                             params['sem_W2_%d_%d' % (e, l)].reshape(128, 1))
            newhs.append(_tc_sem_b(sums, convs[e][0], convs[e][1], relu=(l == 0)))
        hs = newhs
    outs = [_tc_out(hs[e][:10000], params['fc_out_W_%d' % e],
                    params['fc_out_b_%d' % e].reshape(1, HID), nd, 10000)
            for e in range(2)]
    return jnp.stack(outs, axis=0)
